# baseline (speedup 1.0000x reference)
import functools

import jax
import jax.numpy as jnp
from jax import lax
from jax.experimental import pallas as pl
from jax.experimental.pallas import tpu as pltpu
from jax.experimental.pallas import tpu_sc as plsc

NC = 2
NS = 16
NW = NC * NS
C = 128
G = 4
S = C * G


@functools.partial(jax.jit, static_argnums=(2, 3))
def _gather_rows(table, idx2d, B, D):
    n_chunks = idx2d.shape[0]
    b_per_w = B // NW
    n_slots = b_per_w // S
    chunks_per_w = n_chunks // NW

    mesh = plsc.VectorSubcoreMesh(
        core_axis_name="c", subcore_axis_name="s",
        num_cores=NC, num_subcores=NS)

    @functools.partial(
        pl.kernel,
        out_type=jax.ShapeDtypeStruct((B, D), jnp.float32),
        mesh=mesh,
        scratch_types=[
            pltpu.VMEM((G, C), jnp.int32),
            pltpu.VMEM((S, D), jnp.float32),
            pltpu.SemaphoreType.DMA,
        ],
        compiler_params=pltpu.CompilerParams(use_tc_tiling_on_sc=False),
    )
    def k(table_hbm, idx_hbm, out_hbm, idx_v, rows_v, gsem):
        wid = lax.axis_index("s") * NC + lax.axis_index("c")
        base = wid * b_per_w
        chunk_base = wid * chunks_per_w

        @pl.loop(0, n_slots)
        def _slot(j):
            pltpu.sync_copy(idx_hbm.at[pl.ds(chunk_base + j * G, G)], idx_v)
            copies = [
                pltpu.async_copy(
                    table_hbm.at[idx_v.at[g]],
                    rows_v.at[pl.ds(g * C, C)],
                    gsem)
                for g in range(G)
            ]
            for cp in copies:
                cp.wait()
            pltpu.sync_copy(rows_v, out_hbm.at[pl.ds(base + j * S, S)])

    return k(table, idx2d)


def kernel(tokens, W_E):
    B0, T = tokens.shape
    V, D = W_E.shape
    B = B0 * T
    idx2d = tokens.reshape(B // C, C)
    out = _gather_rows(W_E, idx2d, B, D)
    return out.reshape(B0, T, D)


# speedup vs baseline: 1.0759x; 1.0759x over previous
import functools

import jax
import jax.numpy as jnp
from jax import lax
from jax.experimental import pallas as pl
from jax.experimental.pallas import tpu as pltpu
from jax.experimental.pallas import tpu_sc as plsc

NC = 2
NS = 16
NW = NC * NS
C = 128
G = 4
S = C * G


@functools.partial(jax.jit, static_argnums=(2, 3))
def _gather_rows(table, idx2d, B, D):
    n_chunks = idx2d.shape[0]
    b_per_w = B // NW
    n_slots = b_per_w // S
    chunks_per_w = n_chunks // NW

    mesh = plsc.VectorSubcoreMesh(
        core_axis_name="c", subcore_axis_name="s",
        num_cores=NC, num_subcores=NS)

    @functools.partial(
        pl.kernel,
        out_type=jax.ShapeDtypeStruct((B, D), jnp.float32),
        mesh=mesh,
        scratch_types=[
            pltpu.VMEM((2, G, C), jnp.int32),
            pltpu.VMEM((2, S, D), jnp.float32),
            pltpu.SemaphoreType.DMA,
            pltpu.SemaphoreType.DMA,
            pltpu.SemaphoreType.DMA,
            pltpu.SemaphoreType.DMA,
        ],
        compiler_params=pltpu.CompilerParams(use_tc_tiling_on_sc=False),
    )
    def k(table_hbm, idx_hbm, out_hbm, idx_v, rows_v, g0, g1, s0, s1):
        gsems = (g0, g1)
        ssems = (s0, s1)
        wid = lax.axis_index("s") * NC + lax.axis_index("c")
        base = wid * b_per_w
        chunk_base = wid * chunks_per_w

        def idx_load(j, b):
            pltpu.sync_copy(idx_hbm.at[pl.ds(chunk_base + j * G, G)],
                            idx_v.at[b])

        def fire_gathers(j, b):
            for g in range(G):
                pltpu.async_copy(table_hbm.at[idx_v.at[b, g]],
                                 rows_v.at[b, pl.ds(g * C, C)], gsems[b])

        def wait_gathers(j, b):
            for g in range(G):
                pltpu.make_async_copy(table_hbm.at[idx_v.at[b, g]],
                                      rows_v.at[b, pl.ds(g * C, C)],
                                      gsems[b]).wait()

        def fire_scatter(j, b):
            pltpu.async_copy(rows_v.at[b], out_hbm.at[pl.ds(base + j * S, S)],
                             ssems[b])

        def wait_scatter(j, b):
            pltpu.make_async_copy(rows_v.at[b],
                                  out_hbm.at[pl.ds(base + j * S, S)],
                                  ssems[b]).wait()

        idx_load(0, 0)
        fire_gathers(0, 0)

        @pl.loop(0, n_slots, step=2)
        def _slot(jo):
            for b in (0, 1):
                j = jo + b
                @pl.when(j + 1 < n_slots)
                def _():
                    idx_load(j + 1, 1 - b)

                    @pl.when(j >= 1)
                    def _():
                        wait_scatter(j - 1, 1 - b)

                    fire_gathers(j + 1, 1 - b)

                wait_gathers(j, b)
                fire_scatter(j, b)

        wait_scatter(n_slots - 2, 0)
        wait_scatter(n_slots - 1, 1)

    return k(table, idx2d)


def kernel(tokens, W_E):
    B0, T = tokens.shape
    V, D = W_E.shape
    B = B0 * T
    idx2d = tokens.reshape(B // C, C)
    out = _gather_rows(W_E, idx2d, B, D)
    return out.reshape(B0, T, D)


# speedup vs baseline: 1.0760x; 1.0001x over previous
import functools

import jax
import jax.numpy as jnp
from jax import lax
from jax.experimental import pallas as pl
from jax.experimental.pallas import tpu as pltpu
from jax.experimental.pallas import tpu_sc as plsc

NC = 2
NS = 16
NW = NC * NS
C = 512
G = 1
S = C * G


@functools.partial(jax.jit, static_argnums=(2, 3))
def _gather_rows(table, idx2d, B, D):
    n_chunks = idx2d.shape[0]
    b_per_w = B // NW
    n_slots = b_per_w // S
    chunks_per_w = n_chunks // NW

    mesh = plsc.VectorSubcoreMesh(
        core_axis_name="c", subcore_axis_name="s",
        num_cores=NC, num_subcores=NS)

    @functools.partial(
        pl.kernel,
        out_type=jax.ShapeDtypeStruct((B, D), jnp.float32),
        mesh=mesh,
        scratch_types=[
            pltpu.VMEM((2, G, C), jnp.int32),
            pltpu.VMEM((2, S, D), jnp.float32),
            pltpu.SemaphoreType.DMA,
            pltpu.SemaphoreType.DMA,
            pltpu.SemaphoreType.DMA,
            pltpu.SemaphoreType.DMA,
        ],
        compiler_params=pltpu.CompilerParams(use_tc_tiling_on_sc=False),
    )
    def k(table_hbm, idx_hbm, out_hbm, idx_v, rows_v, g0, g1, s0, s1):
        gsems = (g0, g1)
        ssems = (s0, s1)
        wid = lax.axis_index("s") * NC + lax.axis_index("c")
        base = wid * b_per_w
        chunk_base = wid * chunks_per_w

        def idx_load(j, b):
            pltpu.sync_copy(idx_hbm.at[pl.ds(chunk_base + j * G, G)],
                            idx_v.at[b])

        def fire_gathers(j, b):
            for g in range(G):
                pltpu.async_copy(table_hbm.at[idx_v.at[b, g]],
                                 rows_v.at[b, pl.ds(g * C, C)], gsems[b])

        def wait_gathers(j, b):
            for g in range(G):
                pltpu.make_async_copy(table_hbm.at[idx_v.at[b, g]],
                                      rows_v.at[b, pl.ds(g * C, C)],
                                      gsems[b]).wait()

        def fire_scatter(j, b):
            pltpu.async_copy(rows_v.at[b], out_hbm.at[pl.ds(base + j * S, S)],
                             ssems[b])

        def wait_scatter(j, b):
            pltpu.make_async_copy(rows_v.at[b],
                                  out_hbm.at[pl.ds(base + j * S, S)],
                                  ssems[b]).wait()

        idx_load(0, 0)
        fire_gathers(0, 0)

        @pl.loop(0, n_slots, step=2)
        def _slot(jo):
            for b in (0, 1):
                j = jo + b
                @pl.when(j + 1 < n_slots)
                def _():
                    idx_load(j + 1, 1 - b)

                    @pl.when(j >= 1)
                    def _():
                        wait_scatter(j - 1, 1 - b)

                    fire_gathers(j + 1, 1 - b)

                wait_gathers(j, b)
                fire_scatter(j, b)

        wait_scatter(n_slots - 2, 0)
        wait_scatter(n_slots - 1, 1)

    return k(table, idx2d)


def kernel(tokens, W_E):
    B0, T = tokens.shape
    V, D = W_E.shape
    B = B0 * T
    idx2d = tokens.reshape(B // C, C)
    out = _gather_rows(W_E, idx2d, B, D)
    return out.reshape(B0, T, D)
